# baseline (speedup 1.0000x reference)
import functools

import jax
import jax.numpy as jnp
from jax import lax
from jax.experimental import pallas as pl
from jax.experimental.pallas import tpu as pltpu
from jax.experimental.pallas import tpu_sc as plsc

NUM_ITEMS = 1000000
EMBED = 64
HIDDEN = 128
BATCH = 16384

ROWS_PER_GROUP = 8
NUM_CORES = 2
NUM_SUBCORES = 16
NW = NUM_CORES * NUM_SUBCORES
B_PER_W = BATCH // NW
CHUNK = 32
NCH = B_PER_W // CHUNK
LANES = 16


def _sc_gather(idx, table):
    mesh = plsc.VectorSubcoreMesh(core_axis_name="c", subcore_axis_name="s")

    @functools.partial(
        pl.kernel,
        mesh=mesh,
        out_type=jax.ShapeDtypeStruct((BATCH, EMBED), jnp.float32),
        scratch_types=[
            pltpu.VMEM((B_PER_W,), jnp.int32),
            pltpu.VMEM((CHUNK, ROWS_PER_GROUP, EMBED), jnp.float32),
            pltpu.VMEM((CHUNK, ROWS_PER_GROUP, EMBED), jnp.float32),
            pltpu.VMEM((CHUNK, EMBED), jnp.float32),
            pltpu.VMEM((CHUNK, EMBED), jnp.float32),
            pltpu.SemaphoreType.DMA,
            pltpu.SemaphoreType.DMA,
        ],
        compiler_params=pltpu.CompilerParams(needs_layout_passes=False),
    )
    def k(idx_hbm, table_hbm, out_hbm, idx_v, g0, g1, r0, r1, s0, s1):
        wid = lax.axis_index("s") * NUM_CORES + lax.axis_index("c")
        base = wid * B_PER_W
        pltpu.sync_copy(idx_hbm.at[pl.ds(base, B_PER_W)], idx_v)
        lane = lax.iota(jnp.int32, LANES)

        gbufs = (g0, g1)
        rbufs = (r0, r1)
        sems = (s0, s1)

        def fire(j, b):
            for t in range(CHUNK // LANES):
                rows = (idx_v[pl.ds(j * CHUNK + t * LANES, LANES)] >> 3) << 3
                for s in range(LANES):
                    pltpu.async_copy(
                        table_hbm.at[
                            pl.ds(pl.multiple_of(rows[s], 8), ROWS_PER_GROUP)
                        ],
                        gbufs[b].at[t * LANES + s],
                        sems[b],
                    )

        def drain(b):
            for s in range(CHUNK):
                pltpu.make_async_copy(
                    table_hbm.at[pl.ds(0, ROWS_PER_GROUP)],
                    gbufs[b].at[s],
                    sems[b],
                ).wait()

        fire(0, 0)
        fire(1, 1)

        @pl.loop(0, NCH // 2)
        def _(i):
            for b in range(2):
                j = i * 2 + b
                drain(b)
                for t in range(CHUNK // LANES):
                    ids = idx_v[pl.ds(j * CHUNK + t * LANES, LANES)]
                    rin = ids & 7
                    slot = lane + t * LANES
                    for c in range(EMBED):
                        cc = jnp.full((LANES,), c, jnp.int32)
                        v = plsc.load_gather(gbufs[b], [slot, rin, cc])
                        plsc.store_scatter(rbufs[b], [slot, cc], v)

                @pl.when(j + 2 < NCH)
                def _():
                    fire(j + 2, b)

                pltpu.sync_copy(
                    rbufs[b], out_hbm.at[pl.ds(base + j * CHUNK, CHUNK)]
                )

    return k(idx, table)


def _tc_mlp(emb, W1, b1, W2, b2):
    BB = 2048

    def body(emb_ref, w1_ref, b1_ref, w2_ref, b2_ref, out_ref):
        e = emb_ref[...]
        h = lax.dot_general(e, w1_ref[...], (((1,), (1,)), ((), ())),
                            preferred_element_type=jnp.float32)
        h = jnp.maximum(h + b1_ref[...], 0.0)
        s = jnp.sum(h * w2_ref[...], axis=1, keepdims=True)
        out_ref[...] = jax.nn.sigmoid(s + b2_ref[0, 0])

    return pl.pallas_call(
        body,
        grid=(BATCH // BB,),
        in_specs=[
            pl.BlockSpec((BB, EMBED), lambda i: (i, 0)),
            pl.BlockSpec((HIDDEN, EMBED), lambda i: (0, 0)),
            pl.BlockSpec((1, HIDDEN), lambda i: (0, 0)),
            pl.BlockSpec((1, HIDDEN), lambda i: (0, 0)),
            pl.BlockSpec(memory_space=pltpu.SMEM),
        ],
        out_specs=pl.BlockSpec((BB, 1), lambda i: (i, 0)),
        out_shape=jax.ShapeDtypeStruct((BATCH, 1), jnp.float32),
    )(emb, W1, b1, W2, b2)


def kernel(item_ids, table, W1, b1, W2, b2):
    idx = item_ids.astype(jnp.int32)
    emb = _sc_gather(idx, table)
    out = _tc_mlp(emb, W1, b1.reshape(1, HIDDEN), W2, b2.reshape(1, 1))
    return out[:, 0]


# speedup vs baseline: 1.6413x; 1.6413x over previous
import functools

import jax
import jax.numpy as jnp
from jax import lax
from jax.experimental import pallas as pl
from jax.experimental.pallas import tpu as pltpu
from jax.experimental.pallas import tpu_sc as plsc

NUM_ITEMS = 1000000
EMBED = 64
HIDDEN = 128
BATCH = 16384

ROWS_PER_GROUP = 8
NUM_CORES = 2
NUM_SUBCORES = 16
NW = NUM_CORES * NUM_SUBCORES
B_PER_W = BATCH // NW
CHUNK = 32
NCH = B_PER_W // CHUNK
LANES = 16


def _sc_gather(idx, table3):
    mesh = plsc.VectorSubcoreMesh(core_axis_name="c", subcore_axis_name="s")

    @functools.partial(
        pl.kernel,
        mesh=mesh,
        out_type=jax.ShapeDtypeStruct((BATCH, 1, EMBED), jnp.float32),
        scratch_types=[
            pltpu.VMEM((NCH, CHUNK), jnp.int32),
            pltpu.VMEM((CHUNK, 1, EMBED), jnp.float32),
            pltpu.VMEM((CHUNK, 1, EMBED), jnp.float32),
            pltpu.SemaphoreType.DMA,
            pltpu.SemaphoreType.DMA,
        ],
        compiler_params=pltpu.CompilerParams(needs_layout_passes=False),
    )
    def k(idx_hbm, table_hbm, out_hbm, idx_v, r0, r1, s0, s1):
        wid = lax.axis_index("s") * NUM_CORES + lax.axis_index("c")
        base = wid * B_PER_W
        pltpu.sync_copy(idx_hbm.at[wid], idx_v)

        rbufs = (r0, r1)
        sems = (s0, s1)

        def fire(j, b):
            pltpu.async_copy(table_hbm.at[idx_v.at[j]], rbufs[b], sems[b])

        def wait(j, b):
            pltpu.make_async_copy(
                table_hbm.at[idx_v.at[j]], rbufs[b], sems[b]
            ).wait()

        fire(0, 0)
        fire(1, 1)

        @pl.loop(0, NCH // 2)
        def _(i):
            for b in range(2):
                j = i * 2 + b
                wait(j, b)

                @pl.when(j + 2 < NCH)
                def _():
                    fire(j + 2, b)

                pltpu.sync_copy(
                    rbufs[b], out_hbm.at[pl.ds(base + j * CHUNK, CHUNK)]
                )

    return k(idx, table3)


def _tc_mlp(emb, W1, b1, W2, b2):
    BB = 2048

    def body(emb_ref, w1_ref, b1_ref, w2_ref, b2_ref, out_ref):
        e = emb_ref[...]
        h = lax.dot_general(e, w1_ref[...], (((1,), (1,)), ((), ())),
                            preferred_element_type=jnp.float32)
        h = jnp.maximum(h + b1_ref[...], 0.0)
        s = jnp.sum(h * w2_ref[...], axis=1, keepdims=True)
        out_ref[...] = jax.nn.sigmoid(s + b2_ref[0, 0])

    return pl.pallas_call(
        body,
        grid=(BATCH // BB,),
        in_specs=[
            pl.BlockSpec((BB, EMBED), lambda i: (i, 0)),
            pl.BlockSpec((HIDDEN, EMBED), lambda i: (0, 0)),
            pl.BlockSpec((1, HIDDEN), lambda i: (0, 0)),
            pl.BlockSpec((1, HIDDEN), lambda i: (0, 0)),
            pl.BlockSpec(memory_space=pltpu.SMEM),
        ],
        out_specs=pl.BlockSpec((BB, 1), lambda i: (i, 0)),
        out_shape=jax.ShapeDtypeStruct((BATCH, 1), jnp.float32),
    )(emb, W1, b1, W2, b2)


def kernel(item_ids, table, W1, b1, W2, b2):
    idx = item_ids.astype(jnp.int32).reshape(NW, NCH, CHUNK)
    table3 = table.reshape(NUM_ITEMS, 1, EMBED)
    emb = _sc_gather(idx, table3).reshape(BATCH, EMBED)
    out = _tc_mlp(emb, W1, b1.reshape(1, HIDDEN), W2, b2.reshape(1, 1))
    return out[:, 0]
